# Initial kernel scaffold; baseline (speedup 1.0000x reference)
import jax
import jax.numpy as jnp
from jax.experimental import pallas as pl


def kernel(x, W, b):
    raise NotImplementedError("write your pallas kernel here")


# speedup vs baseline: 1.7681x; 1.7681x over previous
import functools

import jax
import jax.numpy as jnp
from jax import lax
from jax.experimental import pallas as pl
from jax.experimental.pallas import tpu as pltpu
from jax.experimental.pallas import tpu_sc as plsc

TOP_K = 2
N_EXPERTS = 16
D_MODEL = 2048
N_TOKENS = 16384

NUM_CORES = 2
NUM_SUBCORES = 16
NUM_WORKERS = NUM_CORES * NUM_SUBCORES
TOK_PER_WORKER = N_TOKENS // NUM_WORKERS
LANES = 16
BLOCKS_PER_WORKER = TOK_PER_WORKER // LANES

_NEG_BIG = -3.0e38


def _proj_body(x_ref, wt_ref, b_ref, out_ref):
    x = x_ref[...]
    wt = wt_ref[...]
    acc = jnp.dot(x, wt, preferred_element_type=jnp.float32)
    out_ref[...] = acc + b_ref[...]


def _project(xf, wt, b2d, block_m):
    grid = (N_TOKENS // block_m,)
    return pl.pallas_call(
        _proj_body,
        grid=grid,
        in_specs=[
            pl.BlockSpec((block_m, D_MODEL), lambda i: (i, 0)),
            pl.BlockSpec((D_MODEL, N_EXPERTS), lambda i: (0, 0)),
            pl.BlockSpec((1, N_EXPERTS), lambda i: (0, 0)),
        ],
        out_specs=pl.BlockSpec((block_m, N_EXPERTS), lambda i: (i, 0)),
        out_shape=jax.ShapeDtypeStruct((N_TOKENS, N_EXPERTS), jnp.float32),
    )(xf, wt, b2d)


def _router_body(logits_hbm, tkw_hbm, tki_hbm, rw_hbm, lg_v, tkw_v, tki_v, rw_v):
    wid = lax.axis_index("s") * NUM_CORES + lax.axis_index("c")
    base = wid * TOK_PER_WORKER

    pltpu.sync_copy(logits_hbm.at[pl.ds(base, TOK_PER_WORKER)], lg_v)

    lane = lax.broadcasted_iota(jnp.int32, (LANES,), 0)

    def block(t, carry):
        toks = t * LANES + lane
        cols = [
            plsc.load_gather(lg_v, [toks, jnp.full((LANES,), e, jnp.int32)])
            for e in range(N_EXPERTS)
        ]
        m1 = cols[0]
        i1 = jnp.zeros((LANES,), jnp.int32)
        for e in range(1, N_EXPERTS):
            gt = cols[e] > m1
            m1 = jnp.where(gt, cols[e], m1)
            i1 = jnp.where(gt, jnp.int32(e), i1)
        m2 = jnp.full((LANES,), _NEG_BIG, jnp.float32)
        i2 = jnp.zeros((LANES,), jnp.int32)
        for e in range(N_EXPERTS):
            cand = jnp.where(i1 == e, _NEG_BIG, cols[e])
            gt = cand > m2
            m2 = jnp.where(gt, cand, m2)
            i2 = jnp.where(gt, jnp.int32(e), i2)
        s1 = 1.0 / (1.0 + jnp.exp(-m1))
        s2 = 1.0 / (1.0 + jnp.exp(-m2))

        zero_i = jnp.zeros((LANES,), jnp.int32)
        one_i = jnp.full((LANES,), 1, jnp.int32)
        plsc.store_scatter(tkw_v, [toks, zero_i], s1)
        plsc.store_scatter(tkw_v, [toks, one_i], s2)
        plsc.store_scatter(tki_v, [toks, zero_i], i1)
        plsc.store_scatter(tki_v, [toks, one_i], i2)
        zf = jnp.zeros((LANES,), jnp.float32)
        for e in range(N_EXPERTS):
            col = jnp.where(i1 == e, s1, jnp.where(i2 == e, s2, zf))
            plsc.store_scatter(rw_v, [toks, jnp.full((LANES,), e, jnp.int32)], col)
        return carry

    lax.fori_loop(0, BLOCKS_PER_WORKER, block, jnp.int32(0))

    pltpu.sync_copy(tkw_v, tkw_hbm.at[pl.ds(base, TOK_PER_WORKER)])
    pltpu.sync_copy(tki_v, tki_hbm.at[pl.ds(base, TOK_PER_WORKER)])
    pltpu.sync_copy(rw_v, rw_hbm.at[pl.ds(base, TOK_PER_WORKER)])


_route = functools.partial(
    pl.kernel,
    out_type=[
        jax.ShapeDtypeStruct((N_TOKENS, TOP_K), jnp.float32),
        jax.ShapeDtypeStruct((N_TOKENS, TOP_K), jnp.int32),
        jax.ShapeDtypeStruct((N_TOKENS, N_EXPERTS), jnp.float32),
    ],
    mesh=plsc.VectorSubcoreMesh(core_axis_name="c", subcore_axis_name="s"),
    scratch_types=[
        pltpu.VMEM((TOK_PER_WORKER, N_EXPERTS), jnp.float32),
        pltpu.VMEM((TOK_PER_WORKER, TOP_K), jnp.float32),
        pltpu.VMEM((TOK_PER_WORKER, TOP_K), jnp.int32),
        pltpu.VMEM((TOK_PER_WORKER, N_EXPERTS), jnp.float32),
    ],
    compiler_params=pltpu.CompilerParams(
        needs_layout_passes=False, use_tc_tiling_on_sc=False
    ),
)(_router_body)


@jax.jit
def kernel(x, W, b):
    xf = x.reshape(N_TOKENS, D_MODEL)
    wt = W.T
    b2d = b.reshape(1, N_EXPERTS)
    logits = _project(xf, wt, b2d, block_m=1024)
    top_k_weight, top_k_idx, router_weight = _route(logits)
    return top_k_weight, top_k_idx, router_weight
